# Initial kernel scaffold; baseline (speedup 1.0000x reference)
import jax
import jax.numpy as jnp
from jax.experimental import pallas as pl


def kernel(x_host, x_flow, edge_index_hf, edge_index_fh, Wl_hf_0, bl_hf_0, Wr_hf_0, Wl_fh_0, bl_fh_0, Wr_fh_0, Wl_hf_1, bl_hf_1, Wr_hf_1, Wl_fh_1, bl_fh_1, Wr_fh_1, Wlin, blin):
    raise NotImplementedError("write your pallas kernel here")


# speedup vs baseline: 4.9231x; 4.9231x over previous
import functools

import jax
import jax.numpy as jnp
from jax import lax
from jax.experimental import pallas as pl
from jax.experimental.pallas import tpu as pltpu
from jax.experimental.pallas import tpu_sc as plsc

N = 50000
D = 128
E = 600000
DIM_OUT = 64

NCH = 8
CW = 16
NPAD = 50176
RPT = NPAD // 16

NG = 8
GW = 128
WE = NG * GW
NWIN = 37
EPT = NWIN * WE
E_PAD = 16 * EPT
WROWS = NWIN * NG

BM = 1024
NBLK = NPAD // BM


WB = RPT // 4


def _agg_body(xparts, srcw, dstw, zrows, zcnt, ones_h,
              agg_out, cnt_out,
              sidx, didx, rows, ones_v, wb, cb, acc_sh, cnt_sh, gsem, ssem):
    cid = lax.axis_index("c")
    sid = lax.axis_index("s")
    rb = sid * RPT
    pltpu.sync_copy(ones_h, ones_v)
    for ci in range(NCH // 2):
        chunk = cid * (NCH // 2) + ci
        pltpu.sync_copy(zrows.at[pl.ds(0, WB)], wb)
        for j in range(4):
            pltpu.sync_copy(wb, acc_sh.at[pl.ds(rb + j * WB, WB)])
        if ci == 0:
            @pl.when(cid == 0)
            def _():
                pltpu.sync_copy(zcnt.at[pl.ds(0, RPT)], cb)
                pltpu.sync_copy(cb, cnt_sh.at[pl.ds(rb, RPT)])
        plsc.subcore_barrier()

        base_row = sid * WROWS

        def wbody(w, carry, *, ci=ci):
            wr = base_row + w * NG
            pltpu.sync_copy(srcw.at[chunk, pl.ds(wr, NG)], sidx)
            pltpu.sync_copy(dstw.at[pl.ds(wr, NG)], didx)
            gds = [pltpu.async_copy(xparts.at[sidx.at[g]], rows.at[g], gsem)
                   for g in range(NG)]
            for d in gds:
                d.wait()
            sds = []
            for g in range(NG):
                sds.append(pltpu.async_copy(rows.at[g], acc_sh.at[didx.at[g]],
                                            ssem, add=True))
            for d in sds:
                d.wait()
            if ci == 0:
                @pl.when(cid == 0)
                def _():
                    cds = [pltpu.async_copy(ones_v, cnt_sh.at[didx.at[g]],
                                            ssem, add=True)
                           for g in range(NG)]
                    for d in cds:
                        d.wait()
            return carry

        lax.fori_loop(0, NWIN, wbody, 0)
        plsc.subcore_barrier()
        for j in range(4):
            pltpu.sync_copy(acc_sh.at[pl.ds(rb + j * WB, WB)], wb)
            pltpu.sync_copy(wb, agg_out.at[chunk, pl.ds(rb + j * WB, WB)])
        if ci == 0:
            @pl.when(cid == 0)
            def _():
                pltpu.sync_copy(cnt_sh.at[pl.ds(rb, RPT)], cb)
                pltpu.sync_copy(cb, cnt_out.at[pl.ds(rb, RPT)])


_agg_call = pl.kernel(
    _agg_body,
    out_type=(jax.ShapeDtypeStruct((NCH, NPAD, CW), jnp.float32),
              jax.ShapeDtypeStruct((NPAD,), jnp.float32)),
    mesh=plsc.VectorSubcoreMesh(core_axis_name="c", subcore_axis_name="s"),
    scratch_types=[
        pltpu.VMEM((NG, GW), jnp.int32),
        pltpu.VMEM((NG, GW), jnp.int32),
        pltpu.VMEM((NG, GW, CW), jnp.float32),
        pltpu.VMEM((GW,), jnp.float32),
        pltpu.VMEM((RPT // 4, CW), jnp.float32),
        pltpu.VMEM((RPT,), jnp.float32),
        pltpu.VMEM_SHARED((NPAD, CW), jnp.float32),
        pltpu.VMEM_SHARED((NPAD,), jnp.float32),
        pltpu.SemaphoreType.DMA,
        pltpu.SemaphoreType.DMA,
    ],
    compiler_params=pltpu.CompilerParams(use_tc_tiling_on_sc=False),
)


def _tc_body(agg_ref, cnt_ref, x_ref, wl_ref, wr_ref, b_ref, *rest, mode):
    if mode == "head":
        wh_ref, bh_ref, o_ref = rest
    else:
        (o_ref,) = rest
    s = jnp.zeros((BM, D), dtype=jnp.float32)
    for c in range(NCH):
        s += jnp.dot(agg_ref[c], wl_ref[c * CW:(c + 1) * CW, :],
                     preferred_element_type=jnp.float32)
    r = 1.0 / jnp.maximum(cnt_ref[...], 1.0)
    z = s * r + jnp.dot(x_ref[...], wr_ref[...],
                        preferred_element_type=jnp.float32) + b_ref[...]
    z = jnp.where(z >= 0, z, 0.01 * z)
    if mode == "plain":
        o_ref[...] = z
    elif mode == "parts":
        for c in range(NCH):
            o_ref[c] = z[:, c * CW:(c + 1) * CW]
    else:
        o_ref[...] = jnp.dot(z, wh_ref[...],
                             preferred_element_type=jnp.float32) + bh_ref[...]


def _make_tc(mode):
    in_specs = [
        pl.BlockSpec((NCH, BM, CW), lambda i: (0, i, 0)),
        pl.BlockSpec((BM, 1), lambda i: (i, 0)),
        pl.BlockSpec((BM, D), lambda i: (i, 0)),
        pl.BlockSpec((D, D), lambda i: (0, 0)),
        pl.BlockSpec((D, D), lambda i: (0, 0)),
        pl.BlockSpec((1, D), lambda i: (0, 0)),
    ]
    if mode == "head":
        in_specs += [pl.BlockSpec((D, DIM_OUT), lambda i: (0, 0)),
                     pl.BlockSpec((1, DIM_OUT), lambda i: (0, 0))]
        out_spec = pl.BlockSpec((BM, DIM_OUT), lambda i: (i, 0))
        out_shape = jax.ShapeDtypeStruct((NPAD, DIM_OUT), jnp.float32)
    elif mode == "parts":
        out_spec = pl.BlockSpec((NCH, BM, CW), lambda i: (0, i, 0))
        out_shape = jax.ShapeDtypeStruct((NCH, NPAD, CW), jnp.float32)
    else:
        out_spec = pl.BlockSpec((BM, D), lambda i: (i, 0))
        out_shape = jax.ShapeDtypeStruct((NPAD, D), jnp.float32)
    return pl.pallas_call(
        functools.partial(_tc_body, mode=mode),
        grid=(NBLK,),
        in_specs=in_specs,
        out_specs=out_spec,
        out_shape=out_shape,
    )


_tc_plain = _make_tc("plain")
_tc_parts = _make_tc("parts")
_tc_head = _make_tc("head")


def _prep_edges(ei):
    src = ei[0].astype(jnp.int32)
    dst = ei[1].astype(jnp.int32)
    pad = E_PAD - E
    j = jnp.arange(pad, dtype=jnp.int32)
    src = jnp.concatenate([src, (j * 37) % N])
    dst = jnp.concatenate([dst, N + j % (NPAD - N)])
    offs = (jnp.arange(NCH, dtype=jnp.int32) * NPAD)[:, None]
    srcw = (src[None, :] + offs).reshape(NCH, E_PAD // GW, GW)
    dstw = dst.reshape(E_PAD // GW, GW)
    return srcw, dstw


def _parts(x):
    xp = jnp.pad(x, ((0, NPAD - N), (0, 0)))
    return xp.reshape(NPAD, NCH, CW).transpose(1, 0, 2).reshape(NCH * NPAD, CW)


def kernel(x_host, x_flow, edge_index_hf, edge_index_fh,
           Wl_hf_0, bl_hf_0, Wr_hf_0, Wl_fh_0, bl_fh_0, Wr_fh_0,
           Wl_hf_1, bl_hf_1, Wr_hf_1, Wl_fh_1, bl_fh_1, Wr_fh_1,
           Wlin, blin):
    srcw_hf, dstw_hf = _prep_edges(edge_index_hf)
    srcw_fh, dstw_fh = _prep_edges(edge_index_fh)
    zrows = jnp.zeros((NPAD, CW), jnp.float32)
    zcnt = jnp.zeros((NPAD,), jnp.float32)
    ones128 = jnp.ones((GW,), jnp.float32)

    xh_parts = _parts(x_host)
    xf_parts = _parts(x_flow)
    xh_pad = jnp.pad(x_host, ((0, NPAD - N), (0, 0)))
    xf_pad = jnp.pad(x_flow, ((0, NPAD - N), (0, 0)))

    agg_hf0, cnt_hf = _agg_call(xh_parts, srcw_hf, dstw_hf, zrows, zcnt, ones128)
    agg_fh0, cnt_fh = _agg_call(xf_parts, srcw_fh, dstw_fh, zrows, zcnt, ones128)
    cnt_hf2 = cnt_hf.reshape(NPAD, 1)
    cnt_fh2 = cnt_fh.reshape(NPAD, 1)

    b = lambda v: v.reshape(1, -1)
    x_flow1 = _tc_plain(agg_hf0, cnt_hf2, xf_pad, Wl_hf_0, Wr_hf_0, b(bl_hf_0))
    xh1_parts = _tc_parts(agg_fh0, cnt_fh2, xh_pad, Wl_fh_0, Wr_fh_0, b(bl_fh_0))

    agg_hf1, _ = _agg_call(xh1_parts.reshape(NCH * NPAD, CW),
                           srcw_hf, dstw_hf, zrows, zcnt, ones128)
    out = _tc_head(agg_hf1, cnt_hf2, x_flow1, Wl_hf_1, Wr_hf_1, b(bl_hf_1),
                   Wlin, b(blin))
    return out[:N]


# speedup vs baseline: 7.1057x; 1.4433x over previous
import functools

import jax
import jax.numpy as jnp
from jax import lax
from jax.experimental import pallas as pl
from jax.experimental.pallas import tpu as pltpu
from jax.experimental.pallas import tpu_sc as plsc

N = 50000
D = 128
E = 600000
DIM_OUT = 64

NCH = 8
CW = 16
NPAD = 50176
RPT = NPAD // 16
WB = RPT // 8

NG = 4
GW = 128
WE = NG * GW
NWIN = 80
EPT = NWIN * WE
E_PAD = 16 * EPT
WROWS = NWIN * NG
NRING = 4

BM = 1024
NBLK = NPAD // BM


def _agg_impl(xparts, srcw, dstw, zrows, zcnt, ones_h,
              agg_out, cnt_out,
              sidx, didx, rows, ones_v, wbufs, zbuf, cbz, acc_sh, cnt_sh,
              gsems, ssems, isems, zsem, wsems, csem):
    cid = lax.axis_index("c")
    sid = lax.axis_index("s")
    rb = sid * RPT
    pltpu.sync_copy(ones_h, ones_v)
    pltpu.sync_copy(zrows.at[pl.ds(0, WB)], zbuf)
    if cnt_out is not None:
        NCW = (E_PAD // 2) // (16 * WE)
        pltpu.sync_copy(zcnt.at[pl.ds(0, RPT)], cbz)
        pltpu.sync_copy(cbz, cnt_sh.at[pl.ds(rb, RPT)])
        plsc.subcore_barrier()
        cbase = cid * (E_PAD // 2 // GW) + sid * (NCW * NG)

        def cbody(w, carry):
            pltpu.sync_copy(dstw.at[pl.ds(cbase + w * NG, NG)], didx.at[0])
            for g in range(NG):
                pltpu.async_copy(ones_v, cnt_sh.at[didx.at[0, g]], csem,
                                 add=True)
            pltpu.make_async_copy(dstw.at[pl.ds(0, NG)], didx.at[1],
                                  csem).wait()
            return carry

        lax.fori_loop(0, NCW, cbody, 0)
        plsc.subcore_barrier()
        pltpu.sync_copy(cnt_sh.at[pl.ds(rb, RPT)], cbz)
        pltpu.sync_copy(cbz, cnt_out.at[cid, pl.ds(rb, RPT)])

    base_row = sid * WROWS

    for ci in range(NCH // 2):
        chunk = cid * (NCH // 2) + ci
        do_cnt = False
        for j in range(8):
            pltpu.async_copy(zbuf, acc_sh.at[pl.ds(rb + j * WB, WB)], zsem)
        pltpu.make_async_copy(zrows.at[pl.ds(0, RPT)],
                              acc_sh.at[pl.ds(rb, RPT)], zsem).wait()
        plsc.subcore_barrier()

        pltpu.async_copy(srcw.at[chunk, pl.ds(base_row, NG)], sidx.at[0],
                         isems[0])
        pltpu.async_copy(dstw.at[pl.ds(base_row, NG)], didx.at[0], isems[0])

        def quad_body(t, carry):
            for k in range(4):
                wv = t * 4 + k
                km2 = (k - 2) % NRING
                km3 = (k - 3) % NRING
                kp1 = (k + 1) % NRING

                @pl.when(jnp.logical_and(wv >= 2, wv < NWIN + 2))
                def _():
                    pltpu.make_async_copy(zrows.at[pl.ds(0, WE)],
                                          rows.at[km2], gsems[km2]).wait()
                    for g in range(NG):
                        pltpu.async_copy(
                            rows.at[km2, pl.ds(g * GW, GW)],
                            acc_sh.at[didx.at[km2, g]],
                            ssems[km2], add=True)
                    if do_cnt:
                        @pl.when(cid == 0)
                        def _():
                            for g in range(NG):
                                pltpu.async_copy(ones_v,
                                                 cnt_sh.at[didx.at[km2, g]],
                                                 ssems[km2], add=True)

                @pl.when(jnp.logical_and(wv >= 3, wv < NWIN + 3))
                def _():
                    pltpu.make_async_copy(zrows.at[pl.ds(0, WE)],
                                          rows.at[km3], ssems[km3]).wait()
                    if do_cnt:
                        @pl.when(cid == 0)
                        def _():
                            pltpu.make_async_copy(
                                zcnt.at[pl.ds(0, NG * GW)],
                                cnt_sh.at[pl.ds(0, NG * GW)],
                                ssems[km3]).wait()

                @pl.when(wv < NWIN)
                def _():
                    pltpu.make_async_copy(srcw.at[chunk, pl.ds(0, NG)],
                                          sidx.at[k], isems[k]).wait()
                    pltpu.make_async_copy(dstw.at[pl.ds(0, NG)],
                                          didx.at[k], isems[k]).wait()
                    for g in range(NG):
                        pltpu.async_copy(
                            xparts.at[sidx.at[k, g]],
                            rows.at[k, pl.ds(g * GW, GW)],
                            gsems[k])

                @pl.when(wv + 1 < NWIN)
                def _():
                    wr = base_row + (wv + 1) * NG
                    pltpu.async_copy(srcw.at[chunk, pl.ds(wr, NG)],
                                     sidx.at[kp1], isems[kp1])
                    pltpu.async_copy(dstw.at[pl.ds(wr, NG)],
                                     didx.at[kp1], isems[kp1])
            return carry

        lax.fori_loop(0, (NWIN + 3 + 3) // 4, quad_body, 0)
        plsc.subcore_barrier()

        for j in range(8):
            p = j % 2
            if j >= 2:
                pltpu.make_async_copy(zrows.at[pl.ds(0, WB)], wbufs.at[p],
                                      wsems[p]).wait()
            pltpu.sync_copy(acc_sh.at[pl.ds(rb + j * WB, WB)], wbufs.at[p])
            pltpu.async_copy(wbufs.at[p],
                             agg_out.at[chunk, pl.ds(rb + j * WB, WB)],
                             wsems[p])
        for p in range(2):
            pltpu.make_async_copy(zrows.at[pl.ds(0, WB)], wbufs.at[p],
                                  wsems[p]).wait()


def _agg_body_c(xparts, srcw, dstw, zrows, zcnt, ones_h, agg_out, cnt_out,
                *scratch):
    _agg_impl(xparts, srcw, dstw, zrows, zcnt, ones_h, agg_out, cnt_out,
              *scratch)


def _agg_body_nc(xparts, srcw, dstw, zrows, zcnt, ones_h, agg_out, *scratch):
    _agg_impl(xparts, srcw, dstw, zrows, zcnt, ones_h, agg_out, None,
              *scratch)


_SCRATCH = [
    pltpu.VMEM((NRING, NG, GW), jnp.int32),
    pltpu.VMEM((NRING, NG, GW), jnp.int32),
    pltpu.VMEM((NRING, WE, CW), jnp.float32),
    pltpu.VMEM((GW,), jnp.float32),
    pltpu.VMEM((2, WB, CW), jnp.float32),
    pltpu.VMEM((WB, CW), jnp.float32),
    pltpu.VMEM((RPT,), jnp.float32),
    pltpu.VMEM_SHARED((NPAD, CW), jnp.float32),
    pltpu.VMEM_SHARED((NPAD,), jnp.float32),
    [pltpu.SemaphoreType.DMA] * NRING,
    [pltpu.SemaphoreType.DMA] * NRING,
    [pltpu.SemaphoreType.DMA] * NRING,
    pltpu.SemaphoreType.DMA,
    [pltpu.SemaphoreType.DMA] * 2,
    pltpu.SemaphoreType.DMA,
]

_agg_call_c = pl.kernel(
    _agg_body_c,
    out_type=(jax.ShapeDtypeStruct((NCH, NPAD, CW), jnp.float32),
              jax.ShapeDtypeStruct((2, NPAD), jnp.float32)),
    mesh=plsc.VectorSubcoreMesh(core_axis_name="c", subcore_axis_name="s"),
    scratch_types=_SCRATCH,
    compiler_params=pltpu.CompilerParams(use_tc_tiling_on_sc=False),
)

_agg_call_nc = pl.kernel(
    _agg_body_nc,
    out_type=jax.ShapeDtypeStruct((NCH, NPAD, CW), jnp.float32),
    mesh=plsc.VectorSubcoreMesh(core_axis_name="c", subcore_axis_name="s"),
    scratch_types=_SCRATCH,
    compiler_params=pltpu.CompilerParams(use_tc_tiling_on_sc=False),
)


def _tc_body(agg_ref, cnt_ref, x_ref, wl_ref, wr_ref, b_ref, *rest, mode):
    if mode == "head":
        wh_ref, bh_ref, o_ref = rest
    else:
        (o_ref,) = rest
    s = jnp.zeros((BM, D), dtype=jnp.float32)
    for c in range(NCH):
        s += jnp.dot(agg_ref[c], wl_ref[c * CW:(c + 1) * CW, :],
                     preferred_element_type=jnp.float32)
    cnt = cnt_ref[0] + cnt_ref[1]
    r = 1.0 / jnp.maximum(cnt, 1.0)
    z = s * r + jnp.dot(x_ref[...], wr_ref[...],
                        preferred_element_type=jnp.float32) + b_ref[...]
    z = jnp.where(z >= 0, z, 0.01 * z)
    if mode == "plain":
        o_ref[...] = z
    elif mode == "parts":
        for c in range(NCH):
            o_ref[c] = z[:, c * CW:(c + 1) * CW]
    else:
        o_ref[...] = jnp.dot(z, wh_ref[...],
                             preferred_element_type=jnp.float32) + bh_ref[...]


def _make_tc(mode):
    in_specs = [
        pl.BlockSpec((NCH, BM, CW), lambda i: (0, i, 0)),
        pl.BlockSpec((2, BM, 1), lambda i: (0, i, 0)),
        pl.BlockSpec((BM, D), lambda i: (i, 0)),
        pl.BlockSpec((D, D), lambda i: (0, 0)),
        pl.BlockSpec((D, D), lambda i: (0, 0)),
        pl.BlockSpec((1, D), lambda i: (0, 0)),
    ]
    if mode == "head":
        in_specs += [pl.BlockSpec((D, DIM_OUT), lambda i: (0, 0)),
                     pl.BlockSpec((1, DIM_OUT), lambda i: (0, 0))]
        out_spec = pl.BlockSpec((BM, DIM_OUT), lambda i: (i, 0))
        out_shape = jax.ShapeDtypeStruct((NPAD, DIM_OUT), jnp.float32)
    elif mode == "parts":
        out_spec = pl.BlockSpec((NCH, BM, CW), lambda i: (0, i, 0))
        out_shape = jax.ShapeDtypeStruct((NCH, NPAD, CW), jnp.float32)
    else:
        out_spec = pl.BlockSpec((BM, D), lambda i: (i, 0))
        out_shape = jax.ShapeDtypeStruct((NPAD, D), jnp.float32)
    return pl.pallas_call(
        functools.partial(_tc_body, mode=mode),
        grid=(NBLK,),
        in_specs=in_specs,
        out_specs=out_spec,
        out_shape=out_shape,
    )


_tc_plain = _make_tc("plain")
_tc_parts = _make_tc("parts")
_tc_head = _make_tc("head")


def _prep_edges(ei):
    src = ei[0].astype(jnp.int32)
    dst = ei[1].astype(jnp.int32)
    pad = E_PAD - E
    j = jnp.arange(pad, dtype=jnp.int32)
    src = jnp.concatenate([src, (j * 37) % N])
    dst = jnp.concatenate([dst, N + j % (NPAD - N)])
    offs = (jnp.arange(NCH, dtype=jnp.int32) * NPAD)[:, None]
    srcw = (src[None, :] + offs).reshape(NCH, E_PAD // GW, GW)
    dstw = dst.reshape(E_PAD // GW, GW)
    return srcw, dstw


def _parts(x):
    xp = jnp.pad(x, ((0, NPAD - N), (0, 0)))
    return xp.reshape(NPAD, NCH, CW).transpose(1, 0, 2).reshape(NCH * NPAD, CW)


def kernel(x_host, x_flow, edge_index_hf, edge_index_fh,
           Wl_hf_0, bl_hf_0, Wr_hf_0, Wl_fh_0, bl_fh_0, Wr_fh_0,
           Wl_hf_1, bl_hf_1, Wr_hf_1, Wl_fh_1, bl_fh_1, Wr_fh_1,
           Wlin, blin):
    srcw_hf, dstw_hf = _prep_edges(edge_index_hf)
    srcw_fh, dstw_fh = _prep_edges(edge_index_fh)
    zrows = jnp.zeros((NPAD, CW), jnp.float32)
    zcnt = jnp.zeros((NPAD,), jnp.float32)
    ones128 = jnp.ones((GW,), jnp.float32)

    xh_parts = _parts(x_host)
    xf_parts = _parts(x_flow)
    xh_pad = jnp.pad(x_host, ((0, NPAD - N), (0, 0)))
    xf_pad = jnp.pad(x_flow, ((0, NPAD - N), (0, 0)))

    agg_hf0, cnt_hf = _agg_call_c(xh_parts, srcw_hf, dstw_hf, zrows, zcnt,
                                  ones128)
    agg_fh0, cnt_fh = _agg_call_c(xf_parts, srcw_fh, dstw_fh, zrows, zcnt,
                                  ones128)
    cnt_hf2 = cnt_hf.reshape(2, NPAD, 1)
    cnt_fh2 = cnt_fh.reshape(2, NPAD, 1)

    b = lambda v: v.reshape(1, -1)
    x_flow1 = _tc_plain(agg_hf0, cnt_hf2, xf_pad, Wl_hf_0, Wr_hf_0, b(bl_hf_0))
    xh1_parts = _tc_parts(agg_fh0, cnt_fh2, xh_pad, Wl_fh_0, Wr_fh_0,
                          b(bl_fh_0))

    agg_hf1 = _agg_call_nc(xh1_parts.reshape(NCH * NPAD, CW),
                           srcw_hf, dstw_hf, zrows, zcnt, ones128)
    out = _tc_head(agg_hf1, cnt_hf2, x_flow1, Wl_hf_1, Wr_hf_1, b(bl_hf_1),
                   Wlin, b(blin))
    return out[:N]
